# baseline (speedup 1.0000x reference)
import functools

import jax
import jax.numpy as jnp
from jax import lax
from jax.experimental import pallas as pl
from jax.experimental.pallas import tpu as pltpu
from jax.experimental.pallas import tpu_sc as plsc

_NC = 2
_NS = 16
_NW = _NC * _NS
_L = 16


def _detile_tc(table_t):
    D, V = table_t.shape
    MAIN = (V // 128) * 128
    TAIL = V - MAIN
    TAILBLK = ((D * TAIL + 127) // 128) * 128
    TOTAL = D * MAIN + TAILBLK

    def body(tab_ref, out_ref, vtail_ref, stg_ref, sem, sem2):
        copies = [
            pltpu.async_copy(
                tab_ref.at[j, pl.ds(0, MAIN)],
                out_ref.at[pl.ds(j * MAIN, MAIN)],
                sem,
            )
            for j in range(D)
        ]
        pltpu.async_copy(tab_ref.at[:, pl.ds(MAIN, TAIL)], vtail_ref, sem2).wait()
        v = vtail_ref[...]
        for r in range(TAILBLK // 128):
            stg_ref[pl.ds(r * 128, 128)] = jnp.concatenate(
                [v[2 * r], v[2 * r + 1]], axis=0
            )
        pltpu.async_copy(
            stg_ref, out_ref.at[pl.ds(D * MAIN, TAILBLK)], sem2
        ).wait()
        for c in copies:
            c.wait()

    return pl.pallas_call(
        body,
        in_specs=[pl.BlockSpec(memory_space=pl.ANY)],
        out_specs=pl.BlockSpec(memory_space=pl.ANY),
        out_shape=jax.ShapeDtypeStruct((TOTAL,), jnp.float32),
        scratch_shapes=[
            pltpu.VMEM((D, TAIL), jnp.float32),
            pltpu.VMEM((TAILBLK,), jnp.float32),
            pltpu.SemaphoreType.DMA,
            pltpu.SemaphoreType.DMA,
        ],
    )(table_t)


def _sc_gather_flat(indices, flat, D, V):
    (B,) = indices.shape
    MAIN = (V // 128) * 128
    TAIL = V - MAIN
    bpw = B // _NW
    mesh = plsc.VectorSubcoreMesh(core_axis_name="c", subcore_axis_name="s")

    @functools.partial(
        pl.kernel,
        mesh=mesh,
        compiler_params=pltpu.CompilerParams(use_tc_tiling_on_sc=False),
        out_type=jax.ShapeDtypeStruct((D, B), jnp.float32),
        scratch_types=[
            pltpu.VMEM((bpw,), jnp.int32),
            pltpu.VMEM((D, bpw), jnp.int32),
            pltpu.VMEM((D, bpw), jnp.float32),
            pltpu.SemaphoreType.DMA,
        ],
    )
    def k(flat_hbm, idx_hbm, out_hbm, idx_v, e_v, cols_v, sem):
        wid = lax.axis_index("s") * _NC + lax.axis_index("c")
        base = wid * bpw
        pltpu.sync_copy(idx_hbm.at[pl.ds(base, bpw)], idx_v)
        for i in range(bpw // _L):
            c = idx_v[pl.ds(i * _L, _L)]
            in_main = c < MAIN
            for j in range(D):
                off = jnp.where(
                    in_main,
                    jnp.full((_L,), j * MAIN, jnp.int32),
                    jnp.full((_L,), D * MAIN + j * TAIL - MAIN, jnp.int32),
                )
                e_v[j, pl.ds(i * _L, _L)] = c + off
        copies = [
            pltpu.async_copy(flat_hbm.at[e_v.at[j]], cols_v.at[j], sem)
            for j in range(D)
        ]
        for c in copies:
            c.wait()
        for j in range(D):
            pltpu.sync_copy(cols_v.at[j], out_hbm.at[j, pl.ds(base, bpw)])

    return k(flat, indices)


def _expmap_body(g_ref, o_ref):
    tx = g_ref[0]
    ty = g_ref[1]
    tz = g_ref[2]
    wx = g_ref[3]
    wy = g_ref[4]
    wz = g_ref[5]
    nrms = wx * wx + wy * wy + wz * wz
    ang = jnp.sqrt(jnp.maximum(nrms, 1e-4))
    inv = 1.0 / ang
    fac1 = inv * jnp.sin(ang)
    fac2 = inv * inv * (1.0 - jnp.cos(ang))
    xx = wx * wx
    yy = wy * wy
    zz = wz * wz
    xy = wx * wy
    xz = wx * wz
    yz = wy * wz
    o_ref[0] = 1.0 + fac2 * (xx - nrms)
    o_ref[1] = fac2 * xy - fac1 * wz
    o_ref[2] = fac2 * xz + fac1 * wy
    o_ref[3] = tx
    o_ref[4] = fac2 * xy + fac1 * wz
    o_ref[5] = 1.0 + fac2 * (yy - nrms)
    o_ref[6] = fac2 * yz - fac1 * wx
    o_ref[7] = ty
    o_ref[8] = fac2 * xz - fac1 * wy
    o_ref[9] = fac2 * yz + fac1 * wx
    o_ref[10] = 1.0 + fac2 * (zz - nrms)
    o_ref[11] = tz


def _expmap_tc(gt):
    _, R, C = gt.shape
    return pl.pallas_call(
        _expmap_body,
        out_shape=jax.ShapeDtypeStruct((12, R, C), jnp.float32),
    )(gt)


def kernel(indices, pose_adjustment):
    B = indices.shape[0]
    V, D = pose_adjustment.shape
    idx = indices.astype(jnp.int32)
    table_t = pose_adjustment.T
    flat = _detile_tc(table_t)
    cols = _sc_gather_flat(idx, flat, D, V)
    out12 = _expmap_tc(cols.reshape(6, B // 128, 128))
    return out12.reshape(12, B).T.reshape(B, 3, 4)


# speedup vs baseline: 14.4223x; 14.4223x over previous
import functools

import jax
import jax.numpy as jnp
from jax import lax
from jax.experimental import pallas as pl
from jax.experimental.pallas import tpu as pltpu
from jax.experimental.pallas import tpu_sc as plsc

_NC = 2
_NS = 16
_NW = _NC * _NS
_L = 16


def _repack_body(t_ref, o_ref):
    D, W = t_ref.shape
    K = W // 128
    v = t_ref[...].reshape(D, K, 128)
    vt = jnp.transpose(v, (1, 0, 2))
    pad = jnp.zeros((K, 8 - D, 128), dtype=vt.dtype)
    o_ref[...] = jnp.concatenate([vt, pad], axis=1).reshape(8 * K, 128)


def _repack_tc(table_t):
    D, V = table_t.shape
    T = (V + 127) // 128
    G = 13
    K = T // G
    return pl.pallas_call(
        _repack_body,
        grid=(G,),
        in_specs=[pl.BlockSpec((D, K * 128), lambda g: (0, g))],
        out_specs=pl.BlockSpec((8 * K, 128), lambda g: (g, 0)),
        out_shape=jax.ShapeDtypeStruct((8 * T, 128), jnp.float32),
    )(table_t)


def _sc_gather_tiled(indices, flat8):
    (B,) = indices.shape
    D = 6
    bpw = B // _NW
    mesh = plsc.VectorSubcoreMesh(core_axis_name="c", subcore_axis_name="s")

    @functools.partial(
        pl.kernel,
        mesh=mesh,
        compiler_params=pltpu.CompilerParams(use_tc_tiling_on_sc=False),
        out_type=jax.ShapeDtypeStruct((D, B), jnp.float32),
        scratch_types=[
            pltpu.VMEM((bpw,), jnp.int32),
            pltpu.VMEM((D, bpw), jnp.int32),
            pltpu.VMEM((D, bpw), jnp.float32),
            pltpu.SemaphoreType.DMA,
        ],
    )
    def k(flat_hbm, idx_hbm, out_hbm, idx_v, e_v, cols_v, sem):
        wid = lax.axis_index("s") * _NC + lax.axis_index("c")
        base = wid * bpw
        pltpu.sync_copy(idx_hbm.at[pl.ds(base, bpw)], idx_v)
        for i in range(bpw // _L):
            c = idx_v[pl.ds(i * _L, _L)]
            e0 = (c >> 7) * 1024 + (c & 127)
            for j in range(D):
                e_v[j, pl.ds(i * _L, _L)] = e0 + (j * 128)
        copies = [
            pltpu.async_copy(flat_hbm.at[e_v.at[j]], cols_v.at[j], sem)
            for j in range(D)
        ]
        for c in copies:
            c.wait()
        for j in range(D):
            pltpu.sync_copy(cols_v.at[j], out_hbm.at[j, pl.ds(base, bpw)])

    return k(flat8, indices)


def _expmap_body(g_ref, o_ref):
    tx = g_ref[0]
    ty = g_ref[1]
    tz = g_ref[2]
    wx = g_ref[3]
    wy = g_ref[4]
    wz = g_ref[5]
    nrms = wx * wx + wy * wy + wz * wz
    ang = jnp.sqrt(jnp.maximum(nrms, 1e-4))
    inv = 1.0 / ang
    fac1 = inv * jnp.sin(ang)
    fac2 = inv * inv * (1.0 - jnp.cos(ang))
    xx = wx * wx
    yy = wy * wy
    zz = wz * wz
    xy = wx * wy
    xz = wx * wz
    yz = wy * wz
    o_ref[0] = 1.0 + fac2 * (xx - nrms)
    o_ref[1] = fac2 * xy - fac1 * wz
    o_ref[2] = fac2 * xz + fac1 * wy
    o_ref[3] = tx
    o_ref[4] = fac2 * xy + fac1 * wz
    o_ref[5] = 1.0 + fac2 * (yy - nrms)
    o_ref[6] = fac2 * yz - fac1 * wx
    o_ref[7] = ty
    o_ref[8] = fac2 * xz - fac1 * wy
    o_ref[9] = fac2 * yz + fac1 * wx
    o_ref[10] = 1.0 + fac2 * (zz - nrms)
    o_ref[11] = tz


def _expmap_tc(gt):
    _, R, C = gt.shape
    return pl.pallas_call(
        _expmap_body,
        out_shape=jax.ShapeDtypeStruct((12, R, C), jnp.float32),
    )(gt)


def kernel(indices, pose_adjustment):
    B = indices.shape[0]
    idx = indices.astype(jnp.int32)
    table_t = pose_adjustment.T
    packed = _repack_tc(table_t)
    flat8 = packed.reshape(-1)
    cols = _sc_gather_tiled(idx, flat8)
    out12 = _expmap_tc(cols.reshape(6, B // 128, 128))
    return out12.reshape(12, B).T.reshape(B, 3, 4)
